# baseline (speedup 1.0000x reference)
import functools

import jax
import jax.numpy as jnp
from jax import lax
from jax.experimental import pallas as pl
from jax.experimental.pallas import tpu as pltpu
from jax.experimental.pallas import tpu_sc as plsc

_LOG2 = 0.6931471805599453

_N = 10000
_D = 128
_E = 320000
_NC = 2
_NS = 16
_NW = _NC * _NS
_CH = 128
_CPT = 80
_CPP = 16
_RING = 2
_EPW = _CH * _CPT
_EPAD = _NW * _EPW
_NDUMP = 768
_NACC = _N + _NDUMP
_RPT = 624
_RTAIL = _N - _RPT * _NS


def _edge_agg_body(h_hbm, src_hbm, dst_hbm, zeros_hbm, out_hbm,
                   src_v, dst_v, *bufs):
    rows = bufs[:_RING]
    acc_sh = bufs[_RING]
    gsem = bufs[_RING + 1:2 * _RING + 1]
    ssem = bufs[2 * _RING + 1:]
    cid = lax.axis_index("c")
    sid = lax.axis_index("s")
    wid = cid * _NS + sid
    rz = sid * _RPT
    pltpu.sync_copy(zeros_hbm.at[pl.ds(rz, _RPT), :],
                    acc_sh.at[pl.ds(rz, _RPT), :])

    @pl.when(sid == 0)
    def _():
        pltpu.sync_copy(zeros_hbm.at[pl.ds(_RPT * _NS, _RTAIL), :],
                        acc_sh.at[pl.ds(_RPT * _NS, _RTAIL), :])

    plsc.subcore_barrier()

    def wait_gather(c, j):
        pltpu.make_async_copy(h_hbm.at[src_v.at[c]], rows[j],
                              gsem[j]).wait()

    def scatter(c, j):
        pltpu.async_copy(rows[j], acc_sh.at[dst_v.at[c]], ssem[j],
                         add=True)

    def wait_scatter(c, j):
        pltpu.make_async_copy(rows[j], acc_sh.at[dst_v.at[c]],
                              ssem[j]).wait()

    def gather(c, j):
        pltpu.async_copy(h_hbm.at[src_v.at[c]], rows[j], gsem[j])

    ngrp = _CPP // _RING
    for p in range(_CPT // _CPP):
        cbase = wid * _CPT + p * _CPP
        pltpu.sync_copy(src_hbm.at[pl.ds(cbase, _CPP), :], src_v)
        pltpu.sync_copy(dst_hbm.at[pl.ds(cbase, _CPP), :], dst_v)
        for j in range(_RING):
            gather(j, j)

        def body(k, carry):
            c0 = _RING * k
            for j in range(_RING):
                wait_gather(c0 + j, j)
                scatter(c0 + j, j)
            for j in range(_RING):
                wait_scatter(c0 + j, j)
                gather(c0 + _RING + j, j)
            return carry

        lax.fori_loop(0, ngrp - 1, body, 0)
        cl = _CPP - _RING
        for j in range(_RING):
            wait_gather(cl + j, j)
            scatter(cl + j, j)
        for j in range(_RING):
            wait_scatter(cl + j, j)

    plsc.subcore_barrier()
    pltpu.sync_copy(acc_sh.at[pl.ds(rz, _RPT), :],
                    out_hbm.at[cid, pl.ds(rz, _RPT), :])

    @pl.when(sid == 0)
    def _():
        pltpu.sync_copy(acc_sh.at[pl.ds(_RPT * _NS, _RTAIL), :],
                        out_hbm.at[cid, pl.ds(_RPT * _NS, _RTAIL), :])


def _edge_agg(h, src_p, dst_p, zeros):
    mesh = plsc.VectorSubcoreMesh(core_axis_name="c", subcore_axis_name="s")
    f = pl.kernel(
        _edge_agg_body, mesh=mesh,
        out_type=jax.ShapeDtypeStruct((_NC, _N, _D), jnp.float32),
        scratch_types=(
            [pltpu.VMEM((_CPP, _CH), jnp.int32)] * 2
            + [pltpu.VMEM((_CH, _D), jnp.float32)] * _RING
            + [pltpu.VMEM_SHARED((_NACC, _D), jnp.float32)]
            + [pltpu.SemaphoreType.DMA] * (2 * _RING)
        ),
    )
    return f(h, src_p, dst_p, zeros)


def _layer_body(h_ref, agg_ref, w1_ref, b1_ref, w2_ref, b2_ref, gam_ref,
                bet_ref, batchf_ref, z_ref, pool_ref):
    n, d = h_ref.shape
    g = pool_ref.shape[0]
    z0 = h_ref[...] + agg_ref[0] + agg_ref[1]
    a = jnp.maximum(
        jnp.dot(z0, w1_ref[...], preferred_element_type=jnp.float32)
        + b1_ref[...], 0.0)
    u = jnp.maximum(
        jnp.dot(a, w2_ref[...], preferred_element_type=jnp.float32)
        + b2_ref[...], 0.0)
    mu = jnp.mean(u, axis=0, keepdims=True)
    var = jnp.mean((u - mu) ** 2, axis=0, keepdims=True)
    zn = (u - mu) * (gam_ref[...] * lax.rsqrt(var + 1e-5)) + bet_ref[...]
    z_ref[...] = zn
    onehot = (batchf_ref[...] == lax.broadcasted_iota(jnp.int32, (n, g), 1)
              .astype(jnp.float32)).astype(jnp.float32)
    pool_ref[...] = lax.dot_general(
        onehot, zn, (((0,), (0,)), ((), ())),
        preferred_element_type=jnp.float32)


def _dense_layer(h, agg, w1, b1, w2, b2, gam, bet, batchf, g):
    n, d = h.shape
    return pl.pallas_call(
        _layer_body,
        out_shape=(jax.ShapeDtypeStruct((n, d), jnp.float32),
                   jax.ShapeDtypeStruct((g, d), jnp.float32)),
    )(h, agg, w1, b1.reshape(1, d), w2, b2.reshape(1, d),
      gam.reshape(1, d), bet.reshape(1, d), batchf)


def _head_body(nblk, y_ref, gdW_ref, gdb_ref, gdWs_ref, gdbs_ref, ldW_ref,
               ldb_ref, ldWs_ref, ldbs_ref, z1_ref, z2_ref, z3_ref,
               batchf_ref, out_ref, genc_scr, acc_scr):
    i = pl.program_id(0)
    nb = z1_ref.shape[0]
    g, emb = genc_scr.shape

    @pl.when(i == 0)
    def _():
        y = y_ref[...]
        hg = y
        for j in range(gdW_ref.shape[0]):
            hg = jnp.maximum(
                jnp.dot(hg, gdW_ref[j], preferred_element_type=jnp.float32)
                + gdb_ref[j], 0.0)
        genc_scr[...] = hg + jnp.dot(
            y, gdWs_ref[...], preferred_element_type=jnp.float32
        ) + gdbs_ref[...]
        acc_scr[0] = 0.0
        acc_scr[1] = 0.0

    t = jnp.concatenate([z1_ref[...], z2_ref[...], z3_ref[...]], axis=1)
    hl = t
    for j in range(ldW_ref.shape[0]):
        hl = jnp.maximum(
            jnp.dot(hl, ldW_ref[j], preferred_element_type=jnp.float32)
            + ldb_ref[j], 0.0)
    l_enc = hl + jnp.dot(
        t, ldWs_ref[...], preferred_element_type=jnp.float32
    ) + ldbs_ref[...]
    res = lax.dot_general(l_enc, genc_scr[...], (((1,), (1,)), ((), ())),
                          preferred_element_type=jnp.float32)
    posm = (batchf_ref[...] == lax.broadcasted_iota(jnp.int32, (nb, g), 1)
            .astype(jnp.float32))
    sp = jnp.maximum(-res, 0.0) + jnp.log(1.0 + jnp.exp(-jnp.abs(res)))
    pos_t = jnp.where(posm, _LOG2 - sp, 0.0)
    neg_t = jnp.where(posm, 0.0, sp + res - _LOG2)
    acc_scr[0] += jnp.sum(pos_t)
    acc_scr[1] += jnp.sum(neg_t)

    @pl.when(i == nblk - 1)
    def _():
        n_total = nb * nblk
        out_ref[0] = acc_scr[1] / (n_total * (g - 1)) - acc_scr[0] / n_total


def _heads_loss(y, gd_W, gd_b, gd_Ws, gd_bs, ld_W, ld_b, ld_Ws, ld_bs,
                z1, z2, z3, batchf):
    n, d = z1.shape
    g, emb = y.shape
    nblk = 10
    nb = n // nblk
    full = lambda shape: pl.BlockSpec(shape, lambda i: (0,) * len(shape))
    out = pl.pallas_call(
        functools.partial(_head_body, nblk),
        grid=(nblk,),
        in_specs=[
            full((g, emb)),
            full(gd_W.shape), full((gd_W.shape[0], 1, emb)), full(gd_Ws.shape),
            full((1, emb)),
            full(ld_W.shape), full((ld_W.shape[0], 1, emb)), full(ld_Ws.shape),
            full((1, emb)),
            pl.BlockSpec((nb, d), lambda i: (i, 0)),
            pl.BlockSpec((nb, d), lambda i: (i, 0)),
            pl.BlockSpec((nb, d), lambda i: (i, 0)),
            pl.BlockSpec((nb, 1), lambda i: (i, 0)),
        ],
        out_specs=pl.BlockSpec(memory_space=pltpu.SMEM),
        out_shape=jax.ShapeDtypeStruct((1,), jnp.float32),
        scratch_shapes=[
            pltpu.VMEM((g, emb), jnp.float32),
            pltpu.SMEM((2,), jnp.float32),
        ],
    )(y, gd_W, gd_b.reshape(-1, 1, emb), gd_Ws, gd_bs.reshape(1, emb),
      ld_W, ld_b.reshape(-1, 1, emb), ld_Ws, ld_bs.reshape(1, emb),
      z1, z2, z3, batchf)
    return out[0]


def kernel(x, edge_index, batch, num_graphs, conv_W1, conv_b1, conv_W2,
           conv_b2, bn_gamma, bn_beta, ld_W, ld_b, ld_Ws, ld_bs, gd_W, gd_b,
           gd_Ws, gd_bs):
    n, d = x.shape
    g = 128
    npad = _EPAD - _E
    pad_i = jnp.arange(npad, dtype=jnp.int32)
    src_p = jnp.concatenate(
        [edge_index[0], pad_i % _N]).reshape(-1, _CH)
    dst_p = jnp.concatenate(
        [edge_index[1], _N + pad_i % _NDUMP]).reshape(-1, _CH)
    batchf = batch.astype(jnp.float32).reshape(n, 1)

    zeros = jnp.zeros((n, d), jnp.float32)
    h = x
    zs = []
    pools = []
    for i in range(conv_W1.shape[0]):
        agg = _edge_agg(h, src_p, dst_p, zeros)
        h, pool = _dense_layer(h, agg, conv_W1[i], conv_b1[i], conv_W2[i],
                               conv_b2[i], bn_gamma[i], bn_beta[i], batchf, g)
        zs.append(h)
        pools.append(pool)
    y = jnp.concatenate(pools, axis=1)
    return _heads_loss(y, gd_W, gd_b, gd_Ws, gd_bs, ld_W, ld_b, ld_Ws, ld_bs,
                       zs[0], zs[1], zs[2], batchf)


# speedup vs baseline: 1.1074x; 1.1074x over previous
import functools

import jax
import jax.numpy as jnp
from jax import lax
from jax.experimental import pallas as pl
from jax.experimental.pallas import tpu as pltpu
from jax.experimental.pallas import tpu_sc as plsc

_LOG2 = 0.6931471805599453

_N = 10000
_D = 128
_E = 320000
_NC = 2
_NS = 16
_NW = _NC * _NS
_CH = 48
_CPT = 216
_CPP = 24
_RING = 6
_EPW = _CH * _CPT
_EPAD = _NW * _EPW
_NDUMP = 768
_NACC = _N + _NDUMP
_RPT = 624
_RTAIL = _N - _RPT * _NS


def _edge_agg_body(h_hbm, src_hbm, dst_hbm, zeros_hbm, out_hbm,
                   src_v, dst_v, *bufs):
    rows = bufs[:_RING]
    acc_sh = bufs[_RING]
    gsem = bufs[_RING + 1:2 * _RING + 1]
    ssem = bufs[2 * _RING + 1:]
    cid = lax.axis_index("c")
    sid = lax.axis_index("s")
    wid = cid * _NS + sid
    rz = sid * _RPT
    pltpu.sync_copy(zeros_hbm.at[pl.ds(rz, _RPT), :],
                    acc_sh.at[pl.ds(rz, _RPT), :])

    @pl.when(sid == 0)
    def _():
        pltpu.sync_copy(zeros_hbm.at[pl.ds(_RPT * _NS, _RTAIL), :],
                        acc_sh.at[pl.ds(_RPT * _NS, _RTAIL), :])

    plsc.subcore_barrier()

    def wait_gather(c, j):
        pltpu.make_async_copy(h_hbm.at[src_v.at[c]], rows[j],
                              gsem[j]).wait()

    def scatter(c, j):
        pltpu.async_copy(rows[j], acc_sh.at[dst_v.at[c]], ssem[j],
                         add=True)

    def wait_scatter(c, j):
        pltpu.make_async_copy(rows[j], acc_sh.at[dst_v.at[c]],
                              ssem[j]).wait()

    def gather(c, j):
        pltpu.async_copy(h_hbm.at[src_v.at[c]], rows[j], gsem[j])

    ngrp = _CPP // _RING
    for p in range(_CPT // _CPP):
        cbase = wid * _CPT + p * _CPP
        pltpu.sync_copy(src_hbm.at[pl.ds(cbase, _CPP), :], src_v)
        pltpu.sync_copy(dst_hbm.at[pl.ds(cbase, _CPP), :], dst_v)
        for j in range(_RING):
            gather(j, j)

        def body(k, carry):
            c0 = _RING * k
            for j in range(_RING):
                wait_gather(c0 + j, j)
                scatter(c0 + j, j)
            for j in range(_RING):
                wait_scatter(c0 + j, j)
                gather(c0 + _RING + j, j)
            return carry

        lax.fori_loop(0, ngrp - 1, body, 0)
        cl = _CPP - _RING
        for j in range(_RING):
            wait_gather(cl + j, j)
            scatter(cl + j, j)
        for j in range(_RING):
            wait_scatter(cl + j, j)

    plsc.subcore_barrier()
    pltpu.sync_copy(acc_sh.at[pl.ds(rz, _RPT), :],
                    out_hbm.at[cid, pl.ds(rz, _RPT), :])

    @pl.when(sid == 0)
    def _():
        pltpu.sync_copy(acc_sh.at[pl.ds(_RPT * _NS, _RTAIL), :],
                        out_hbm.at[cid, pl.ds(_RPT * _NS, _RTAIL), :])


def _edge_agg(h, src_p, dst_p, zeros):
    mesh = plsc.VectorSubcoreMesh(core_axis_name="c", subcore_axis_name="s")
    f = pl.kernel(
        _edge_agg_body, mesh=mesh,
        out_type=jax.ShapeDtypeStruct((_NC, _N, _D), jnp.float32),
        scratch_types=(
            [pltpu.VMEM((_CPP, _CH), jnp.int32)] * 2
            + [pltpu.VMEM((_CH, _D), jnp.float32)] * _RING
            + [pltpu.VMEM_SHARED((_NACC, _D), jnp.float32)]
            + [pltpu.SemaphoreType.DMA] * (2 * _RING)
        ),
    )
    return f(h, src_p, dst_p, zeros)


def _layer_body(h_ref, agg_ref, w1_ref, b1_ref, w2_ref, b2_ref, gam_ref,
                bet_ref, batchf_ref, z_ref, pool_ref):
    n, d = h_ref.shape
    g = pool_ref.shape[0]
    z0 = h_ref[...] + agg_ref[0] + agg_ref[1]
    a = jnp.maximum(
        jnp.dot(z0, w1_ref[...], preferred_element_type=jnp.float32)
        + b1_ref[...], 0.0)
    u = jnp.maximum(
        jnp.dot(a, w2_ref[...], preferred_element_type=jnp.float32)
        + b2_ref[...], 0.0)
    mu = jnp.mean(u, axis=0, keepdims=True)
    var = jnp.mean((u - mu) ** 2, axis=0, keepdims=True)
    zn = (u - mu) * (gam_ref[...] * lax.rsqrt(var + 1e-5)) + bet_ref[...]
    z_ref[...] = zn
    onehot = (batchf_ref[...] == lax.broadcasted_iota(jnp.int32, (n, g), 1)
              .astype(jnp.float32)).astype(jnp.float32)
    pool_ref[...] = lax.dot_general(
        onehot, zn, (((0,), (0,)), ((), ())),
        preferred_element_type=jnp.float32)


def _dense_layer(h, agg, w1, b1, w2, b2, gam, bet, batchf, g):
    n, d = h.shape
    return pl.pallas_call(
        _layer_body,
        out_shape=(jax.ShapeDtypeStruct((n, d), jnp.float32),
                   jax.ShapeDtypeStruct((g, d), jnp.float32)),
    )(h, agg, w1, b1.reshape(1, d), w2, b2.reshape(1, d),
      gam.reshape(1, d), bet.reshape(1, d), batchf)


def _head_body(nblk, y_ref, gdW_ref, gdb_ref, gdWs_ref, gdbs_ref, ldW_ref,
               ldb_ref, ldWs_ref, ldbs_ref, z1_ref, z2_ref, z3_ref,
               batchf_ref, out_ref, genc_scr, acc_scr):
    i = pl.program_id(0)
    nb = z1_ref.shape[0]
    g, emb = genc_scr.shape

    @pl.when(i == 0)
    def _():
        y = y_ref[...]
        hg = y
        for j in range(gdW_ref.shape[0]):
            hg = jnp.maximum(
                jnp.dot(hg, gdW_ref[j], preferred_element_type=jnp.float32)
                + gdb_ref[j], 0.0)
        genc_scr[...] = hg + jnp.dot(
            y, gdWs_ref[...], preferred_element_type=jnp.float32
        ) + gdbs_ref[...]
        acc_scr[0] = 0.0
        acc_scr[1] = 0.0

    t = jnp.concatenate([z1_ref[...], z2_ref[...], z3_ref[...]], axis=1)
    hl = t
    for j in range(ldW_ref.shape[0]):
        hl = jnp.maximum(
            jnp.dot(hl, ldW_ref[j], preferred_element_type=jnp.float32)
            + ldb_ref[j], 0.0)
    l_enc = hl + jnp.dot(
        t, ldWs_ref[...], preferred_element_type=jnp.float32
    ) + ldbs_ref[...]
    res = lax.dot_general(l_enc, genc_scr[...], (((1,), (1,)), ((), ())),
                          preferred_element_type=jnp.float32)
    posm = (batchf_ref[...] == lax.broadcasted_iota(jnp.int32, (nb, g), 1)
            .astype(jnp.float32))
    sp = jnp.maximum(-res, 0.0) + jnp.log(1.0 + jnp.exp(-jnp.abs(res)))
    pos_t = jnp.where(posm, _LOG2 - sp, 0.0)
    neg_t = jnp.where(posm, 0.0, sp + res - _LOG2)
    acc_scr[0] += jnp.sum(pos_t)
    acc_scr[1] += jnp.sum(neg_t)

    @pl.when(i == nblk - 1)
    def _():
        n_total = nb * nblk
        out_ref[0] = acc_scr[1] / (n_total * (g - 1)) - acc_scr[0] / n_total


def _heads_loss(y, gd_W, gd_b, gd_Ws, gd_bs, ld_W, ld_b, ld_Ws, ld_bs,
                z1, z2, z3, batchf):
    n, d = z1.shape
    g, emb = y.shape
    nblk = 10
    nb = n // nblk
    full = lambda shape: pl.BlockSpec(shape, lambda i: (0,) * len(shape))
    out = pl.pallas_call(
        functools.partial(_head_body, nblk),
        grid=(nblk,),
        in_specs=[
            full((g, emb)),
            full(gd_W.shape), full((gd_W.shape[0], 1, emb)), full(gd_Ws.shape),
            full((1, emb)),
            full(ld_W.shape), full((ld_W.shape[0], 1, emb)), full(ld_Ws.shape),
            full((1, emb)),
            pl.BlockSpec((nb, d), lambda i: (i, 0)),
            pl.BlockSpec((nb, d), lambda i: (i, 0)),
            pl.BlockSpec((nb, d), lambda i: (i, 0)),
            pl.BlockSpec((nb, 1), lambda i: (i, 0)),
        ],
        out_specs=pl.BlockSpec(memory_space=pltpu.SMEM),
        out_shape=jax.ShapeDtypeStruct((1,), jnp.float32),
        scratch_shapes=[
            pltpu.VMEM((g, emb), jnp.float32),
            pltpu.SMEM((2,), jnp.float32),
        ],
    )(y, gd_W, gd_b.reshape(-1, 1, emb), gd_Ws, gd_bs.reshape(1, emb),
      ld_W, ld_b.reshape(-1, 1, emb), ld_Ws, ld_bs.reshape(1, emb),
      z1, z2, z3, batchf)
    return out[0]


def kernel(x, edge_index, batch, num_graphs, conv_W1, conv_b1, conv_W2,
           conv_b2, bn_gamma, bn_beta, ld_W, ld_b, ld_Ws, ld_bs, gd_W, gd_b,
           gd_Ws, gd_bs):
    n, d = x.shape
    g = 128
    npad = _EPAD - _E
    pad_i = jnp.arange(npad, dtype=jnp.int32)
    src_p = jnp.concatenate(
        [edge_index[0], pad_i % _N]).reshape(-1, _CH)
    dst_p = jnp.concatenate(
        [edge_index[1], _N + pad_i % _NDUMP]).reshape(-1, _CH)
    batchf = batch.astype(jnp.float32).reshape(n, 1)

    zeros = jnp.zeros((n, d), jnp.float32)
    h = x
    zs = []
    pools = []
    for i in range(conv_W1.shape[0]):
        agg = _edge_agg(h, src_p, dst_p, zeros)
        h, pool = _dense_layer(h, agg, conv_W1[i], conv_b1[i], conv_W2[i],
                               conv_b2[i], bn_gamma[i], bn_beta[i], batchf, g)
        zs.append(h)
        pools.append(pool)
    y = jnp.concatenate(pools, axis=1)
    return _heads_loss(y, gd_W, gd_b, gd_Ws, gd_bs, ld_W, ld_b, ld_Ws, ld_bs,
                       zs[0], zs[1], zs[2], batchf)


# speedup vs baseline: 1.1277x; 1.0183x over previous
import functools

import jax
import jax.numpy as jnp
from jax import lax
from jax.experimental import pallas as pl
from jax.experimental.pallas import tpu as pltpu
from jax.experimental.pallas import tpu_sc as plsc

_LOG2 = 0.6931471805599453

_N = 10000
_D = 128
_E = 320000
_NC = 2
_NS = 16
_NW = _NC * _NS
_CH = 32
_CPT = 320
_CPP = 40
_RING = 8
_EPW = _CH * _CPT
_EPAD = _NW * _EPW
_NDUMP = 768
_NACC = _N + _NDUMP
_RPT = 624
_RTAIL = _N - _RPT * _NS


def _edge_agg_body(h_hbm, src_hbm, dst_hbm, zeros_hbm, out_hbm,
                   src_v, dst_v, *bufs):
    rows = bufs[:_RING]
    acc_sh = bufs[_RING]
    gsem = bufs[_RING + 1:2 * _RING + 1]
    ssem = bufs[2 * _RING + 1:]
    cid = lax.axis_index("c")
    sid = lax.axis_index("s")
    wid = cid * _NS + sid
    rz = sid * _RPT
    pltpu.sync_copy(zeros_hbm.at[pl.ds(rz, _RPT), :],
                    acc_sh.at[pl.ds(rz, _RPT), :])

    @pl.when(sid == 0)
    def _():
        pltpu.sync_copy(zeros_hbm.at[pl.ds(_RPT * _NS, _RTAIL), :],
                        acc_sh.at[pl.ds(_RPT * _NS, _RTAIL), :])

    plsc.subcore_barrier()

    def wait_gather(c, j):
        pltpu.make_async_copy(h_hbm.at[src_v.at[c]], rows[j],
                              gsem[j]).wait()

    def scatter(c, j):
        pltpu.async_copy(rows[j], acc_sh.at[dst_v.at[c]], ssem[j],
                         add=True)

    def wait_scatter(c, j):
        pltpu.make_async_copy(rows[j], acc_sh.at[dst_v.at[c]],
                              ssem[j]).wait()

    def gather(c, j):
        pltpu.async_copy(h_hbm.at[src_v.at[c]], rows[j], gsem[j])

    ngrp = _CPP // _RING
    for p in range(_CPT // _CPP):
        cbase = wid * _CPT + p * _CPP
        pltpu.sync_copy(src_hbm.at[pl.ds(cbase, _CPP), :], src_v)
        pltpu.sync_copy(dst_hbm.at[pl.ds(cbase, _CPP), :], dst_v)
        for j in range(_RING):
            gather(j, j)

        def body(k, carry):
            c0 = _RING * k
            for j in range(_RING):
                wait_gather(c0 + j, j)
                scatter(c0 + j, j)
            for j in range(_RING):
                wait_scatter(c0 + j, j)
                gather(c0 + _RING + j, j)
            return carry

        lax.fori_loop(0, ngrp - 1, body, 0)
        cl = _CPP - _RING
        for j in range(_RING):
            wait_gather(cl + j, j)
            scatter(cl + j, j)
        for j in range(_RING):
            wait_scatter(cl + j, j)

    plsc.subcore_barrier()
    pltpu.sync_copy(acc_sh.at[pl.ds(rz, _RPT), :],
                    out_hbm.at[cid, pl.ds(rz, _RPT), :])

    @pl.when(sid == 0)
    def _():
        pltpu.sync_copy(acc_sh.at[pl.ds(_RPT * _NS, _RTAIL), :],
                        out_hbm.at[cid, pl.ds(_RPT * _NS, _RTAIL), :])


def _edge_agg(h, src_p, dst_p, zeros):
    mesh = plsc.VectorSubcoreMesh(core_axis_name="c", subcore_axis_name="s")
    f = pl.kernel(
        _edge_agg_body, mesh=mesh,
        out_type=jax.ShapeDtypeStruct((_NC, _N, _D), jnp.float32),
        scratch_types=(
            [pltpu.VMEM((_CPP, _CH), jnp.int32)] * 2
            + [pltpu.VMEM((_CH, _D), jnp.float32)] * _RING
            + [pltpu.VMEM_SHARED((_NACC, _D), jnp.float32)]
            + [pltpu.SemaphoreType.DMA] * (2 * _RING)
        ),
    )
    return f(h, src_p, dst_p, zeros)


def _layer_body(h_ref, agg_ref, w1_ref, b1_ref, w2_ref, b2_ref, gam_ref,
                bet_ref, batchf_ref, z_ref, pool_ref):
    n, d = h_ref.shape
    g = pool_ref.shape[0]
    z0 = h_ref[...] + agg_ref[0] + agg_ref[1]
    a = jnp.maximum(
        jnp.dot(z0, w1_ref[...], preferred_element_type=jnp.float32)
        + b1_ref[...], 0.0)
    u = jnp.maximum(
        jnp.dot(a, w2_ref[...], preferred_element_type=jnp.float32)
        + b2_ref[...], 0.0)
    mu = jnp.mean(u, axis=0, keepdims=True)
    var = jnp.mean((u - mu) ** 2, axis=0, keepdims=True)
    zn = (u - mu) * (gam_ref[...] * lax.rsqrt(var + 1e-5)) + bet_ref[...]
    z_ref[...] = zn
    onehot = (batchf_ref[...] == lax.broadcasted_iota(jnp.int32, (n, g), 1)
              .astype(jnp.float32)).astype(jnp.float32)
    pool_ref[...] = lax.dot_general(
        onehot, zn, (((0,), (0,)), ((), ())),
        preferred_element_type=jnp.float32)


def _dense_layer(h, agg, w1, b1, w2, b2, gam, bet, batchf, g):
    n, d = h.shape
    return pl.pallas_call(
        _layer_body,
        out_shape=(jax.ShapeDtypeStruct((n, d), jnp.float32),
                   jax.ShapeDtypeStruct((g, d), jnp.float32)),
    )(h, agg, w1, b1.reshape(1, d), w2, b2.reshape(1, d),
      gam.reshape(1, d), bet.reshape(1, d), batchf)


def _head_body(nblk, y_ref, gdW_ref, gdb_ref, gdWs_ref, gdbs_ref, ldW_ref,
               ldb_ref, ldWs_ref, ldbs_ref, z1_ref, z2_ref, z3_ref,
               batchf_ref, out_ref, genc_scr, acc_scr):
    i = pl.program_id(0)
    nb = z1_ref.shape[0]
    g, emb = genc_scr.shape

    @pl.when(i == 0)
    def _():
        y = y_ref[...]
        hg = y
        for j in range(gdW_ref.shape[0]):
            hg = jnp.maximum(
                jnp.dot(hg, gdW_ref[j], preferred_element_type=jnp.float32)
                + gdb_ref[j], 0.0)
        genc_scr[...] = hg + jnp.dot(
            y, gdWs_ref[...], preferred_element_type=jnp.float32
        ) + gdbs_ref[...]
        acc_scr[0] = 0.0
        acc_scr[1] = 0.0

    t = jnp.concatenate([z1_ref[...], z2_ref[...], z3_ref[...]], axis=1)
    hl = t
    for j in range(ldW_ref.shape[0]):
        hl = jnp.maximum(
            jnp.dot(hl, ldW_ref[j], preferred_element_type=jnp.float32)
            + ldb_ref[j], 0.0)
    l_enc = hl + jnp.dot(
        t, ldWs_ref[...], preferred_element_type=jnp.float32
    ) + ldbs_ref[...]
    res = lax.dot_general(l_enc, genc_scr[...], (((1,), (1,)), ((), ())),
                          preferred_element_type=jnp.float32)
    posm = (batchf_ref[...] == lax.broadcasted_iota(jnp.int32, (nb, g), 1)
            .astype(jnp.float32))
    sp = jnp.maximum(-res, 0.0) + jnp.log(1.0 + jnp.exp(-jnp.abs(res)))
    pos_t = jnp.where(posm, _LOG2 - sp, 0.0)
    neg_t = jnp.where(posm, 0.0, sp + res - _LOG2)
    acc_scr[0] += jnp.sum(pos_t)
    acc_scr[1] += jnp.sum(neg_t)

    @pl.when(i == nblk - 1)
    def _():
        n_total = nb * nblk
        out_ref[0] = acc_scr[1] / (n_total * (g - 1)) - acc_scr[0] / n_total


def _heads_loss(y, gd_W, gd_b, gd_Ws, gd_bs, ld_W, ld_b, ld_Ws, ld_bs,
                z1, z2, z3, batchf):
    n, d = z1.shape
    g, emb = y.shape
    nblk = 10
    nb = n // nblk
    full = lambda shape: pl.BlockSpec(shape, lambda i: (0,) * len(shape))
    out = pl.pallas_call(
        functools.partial(_head_body, nblk),
        grid=(nblk,),
        in_specs=[
            full((g, emb)),
            full(gd_W.shape), full((gd_W.shape[0], 1, emb)), full(gd_Ws.shape),
            full((1, emb)),
            full(ld_W.shape), full((ld_W.shape[0], 1, emb)), full(ld_Ws.shape),
            full((1, emb)),
            pl.BlockSpec((nb, d), lambda i: (i, 0)),
            pl.BlockSpec((nb, d), lambda i: (i, 0)),
            pl.BlockSpec((nb, d), lambda i: (i, 0)),
            pl.BlockSpec((nb, 1), lambda i: (i, 0)),
        ],
        out_specs=pl.BlockSpec(memory_space=pltpu.SMEM),
        out_shape=jax.ShapeDtypeStruct((1,), jnp.float32),
        scratch_shapes=[
            pltpu.VMEM((g, emb), jnp.float32),
            pltpu.SMEM((2,), jnp.float32),
        ],
    )(y, gd_W, gd_b.reshape(-1, 1, emb), gd_Ws, gd_bs.reshape(1, emb),
      ld_W, ld_b.reshape(-1, 1, emb), ld_Ws, ld_bs.reshape(1, emb),
      z1, z2, z3, batchf)
    return out[0]


def kernel(x, edge_index, batch, num_graphs, conv_W1, conv_b1, conv_W2,
           conv_b2, bn_gamma, bn_beta, ld_W, ld_b, ld_Ws, ld_bs, gd_W, gd_b,
           gd_Ws, gd_bs):
    n, d = x.shape
    g = 128
    npad = _EPAD - _E
    pad_i = jnp.arange(npad, dtype=jnp.int32)
    src_p = jnp.concatenate(
        [edge_index[0], pad_i % _N]).reshape(-1, _CH)
    dst_p = jnp.concatenate(
        [edge_index[1], _N + pad_i % _NDUMP]).reshape(-1, _CH)
    batchf = batch.astype(jnp.float32).reshape(n, 1)

    zeros = jnp.zeros((n, d), jnp.float32)
    h = x
    zs = []
    pools = []
    for i in range(conv_W1.shape[0]):
        agg = _edge_agg(h, src_p, dst_p, zeros)
        h, pool = _dense_layer(h, agg, conv_W1[i], conv_b1[i], conv_W2[i],
                               conv_b2[i], bn_gamma[i], bn_beta[i], batchf, g)
        zs.append(h)
        pools.append(pool)
    y = jnp.concatenate(pools, axis=1)
    return _heads_loss(y, gd_W, gd_b, gd_Ws, gd_bs, ld_W, ld_b, ld_Ws, ld_bs,
                       zs[0], zs[1], zs[2], batchf)


# speedup vs baseline: 1.2276x; 1.0886x over previous
import functools

import jax
import jax.numpy as jnp
from jax import lax
from jax.experimental import pallas as pl
from jax.experimental.pallas import tpu as pltpu
from jax.experimental.pallas import tpu_sc as plsc

_LOG2 = 0.6931471805599453

_N = 10000
_D = 128
_E = 320000
_NC = 2
_NS = 16
_NW = _NC * _NS
_CH = 64
_CPT = 160
_CPP = 16
_RING = 4
_EPW = _CH * _CPT
_EPAD = _NW * _EPW
_NDUMP = 768
_NACC = _N + _NDUMP
_RPT = 624
_RTAIL = _N - _RPT * _NS


def _edge_agg_body(h_hbm, src_hbm, dst_hbm, zeros_hbm, out_hbm,
                   src_a, dst_a, src_b, dst_b, *bufs):
    rows = bufs[:_RING]
    acc_sh = bufs[_RING]
    isem = bufs[_RING + 1]
    gsem = bufs[_RING + 2:2 * _RING + 2]
    ssem = bufs[2 * _RING + 2:]
    idx = ((src_a, dst_a), (src_b, dst_b))
    cid = lax.axis_index("c")
    sid = lax.axis_index("s")
    wid = cid * _NS + sid
    rz = sid * _RPT
    pltpu.sync_copy(zeros_hbm.at[pl.ds(rz, _RPT), :],
                    acc_sh.at[pl.ds(rz, _RPT), :])

    @pl.when(sid == 0)
    def _():
        pltpu.sync_copy(zeros_hbm.at[pl.ds(_RPT * _NS, _RTAIL), :],
                        acc_sh.at[pl.ds(_RPT * _NS, _RTAIL), :])

    plsc.subcore_barrier()

    def wait_gather(sv, c, j):
        pltpu.make_async_copy(h_hbm.at[sv.at[c]], rows[j], gsem[j]).wait()

    def scatter(dv, c, j):
        pltpu.async_copy(rows[j], acc_sh.at[dv.at[c]], ssem[j], add=True)

    def wait_scatter(dv, c, j):
        pltpu.make_async_copy(rows[j], acc_sh.at[dv.at[c]], ssem[j]).wait()

    def gather(sv, c, j):
        pltpu.async_copy(h_hbm.at[sv.at[c]], rows[j], gsem[j])

    def load_idx(p, sync):
        sv, dv = idx[p % 2]
        cbase = wid * _CPT + p * _CPP
        if sync:
            pltpu.sync_copy(src_hbm.at[pl.ds(cbase, _CPP), :], sv)
            pltpu.sync_copy(dst_hbm.at[pl.ds(cbase, _CPP), :], dv)
        else:
            pltpu.async_copy(src_hbm.at[pl.ds(cbase, _CPP), :], sv, isem)
            pltpu.async_copy(dst_hbm.at[pl.ds(cbase, _CPP), :], dv, isem)

    def wait_idx(p):
        sv, dv = idx[p % 2]
        cbase = wid * _CPT + p * _CPP
        pltpu.make_async_copy(src_hbm.at[pl.ds(cbase, _CPP), :], sv,
                              isem).wait()
        pltpu.make_async_copy(dst_hbm.at[pl.ds(cbase, _CPP), :], dv,
                              isem).wait()

    nphase = _CPT // _CPP
    ngrp = _CPP // _RING
    load_idx(0, sync=True)
    for j in range(_RING):
        gather(idx[0][0], j, j)
    for p in range(nphase):
        sv, dv = idx[p % 2]
        if p + 1 < nphase:
            load_idx(p + 1, sync=False)

        def body(k, carry):
            c0 = _RING * k
            for j in range(_RING):
                wait_gather(sv, c0 + j, j)
                scatter(dv, c0 + j, j)
            for j in range(_RING):
                wait_scatter(dv, c0 + j, j)
                gather(sv, c0 + _RING + j, j)
            return carry

        lax.fori_loop(0, ngrp - 1, body, 0)
        cl = _CPP - _RING
        for j in range(_RING):
            wait_gather(sv, cl + j, j)
            scatter(dv, cl + j, j)
        if p + 1 < nphase:
            wait_idx(p + 1)
            nsv = idx[(p + 1) % 2][0]
            for j in range(_RING):
                wait_scatter(dv, cl + j, j)
                gather(nsv, j, j)
        else:
            for j in range(_RING):
                wait_scatter(dv, cl + j, j)

    plsc.subcore_barrier()
    pltpu.sync_copy(acc_sh.at[pl.ds(rz, _RPT), :],
                    out_hbm.at[cid, pl.ds(rz, _RPT), :])

    @pl.when(sid == 0)
    def _():
        pltpu.sync_copy(acc_sh.at[pl.ds(_RPT * _NS, _RTAIL), :],
                        out_hbm.at[cid, pl.ds(_RPT * _NS, _RTAIL), :])


def _edge_agg(h, src_p, dst_p, zeros):
    mesh = plsc.VectorSubcoreMesh(core_axis_name="c", subcore_axis_name="s")
    f = pl.kernel(
        _edge_agg_body, mesh=mesh,
        out_type=jax.ShapeDtypeStruct((_NC, _N, _D), jnp.float32),
        scratch_types=(
            [pltpu.VMEM((_CPP, _CH), jnp.int32)] * 4
            + [pltpu.VMEM((_CH, _D), jnp.float32)] * _RING
            + [pltpu.VMEM_SHARED((_NACC, _D), jnp.float32)]
            + [pltpu.SemaphoreType.DMA] * (1 + 2 * _RING)
        ),
    )
    return f(h, src_p, dst_p, zeros)


def _layer_body(h_ref, agg_ref, w1_ref, b1_ref, w2_ref, b2_ref, gam_ref,
                bet_ref, batchf_ref, z_ref, pool_ref):
    n, d = h_ref.shape
    g = pool_ref.shape[0]
    z0 = h_ref[...] + agg_ref[0] + agg_ref[1]
    a = jnp.maximum(
        jnp.dot(z0, w1_ref[...], preferred_element_type=jnp.float32)
        + b1_ref[...], 0.0)
    u = jnp.maximum(
        jnp.dot(a, w2_ref[...], preferred_element_type=jnp.float32)
        + b2_ref[...], 0.0)
    mu = jnp.mean(u, axis=0, keepdims=True)
    var = jnp.mean((u - mu) ** 2, axis=0, keepdims=True)
    zn = (u - mu) * (gam_ref[...] * lax.rsqrt(var + 1e-5)) + bet_ref[...]
    z_ref[...] = zn
    onehot = (batchf_ref[...] == lax.broadcasted_iota(jnp.int32, (n, g), 1)
              .astype(jnp.float32)).astype(jnp.float32)
    pool_ref[...] = lax.dot_general(
        onehot, zn, (((0,), (0,)), ((), ())),
        preferred_element_type=jnp.float32)


def _dense_layer(h, agg, w1, b1, w2, b2, gam, bet, batchf, g):
    n, d = h.shape
    return pl.pallas_call(
        _layer_body,
        out_shape=(jax.ShapeDtypeStruct((n, d), jnp.float32),
                   jax.ShapeDtypeStruct((g, d), jnp.float32)),
    )(h, agg, w1, b1.reshape(1, d), w2, b2.reshape(1, d),
      gam.reshape(1, d), bet.reshape(1, d), batchf)


def _head_body(nblk, y_ref, gdW_ref, gdb_ref, gdWs_ref, gdbs_ref, ldW_ref,
               ldb_ref, ldWs_ref, ldbs_ref, z1_ref, z2_ref, z3_ref,
               batchf_ref, out_ref, genc_scr, acc_scr):
    i = pl.program_id(0)
    nb = z1_ref.shape[0]
    g, emb = genc_scr.shape

    @pl.when(i == 0)
    def _():
        y = y_ref[...]
        hg = y
        for j in range(gdW_ref.shape[0]):
            hg = jnp.maximum(
                jnp.dot(hg, gdW_ref[j], preferred_element_type=jnp.float32)
                + gdb_ref[j], 0.0)
        genc_scr[...] = hg + jnp.dot(
            y, gdWs_ref[...], preferred_element_type=jnp.float32
        ) + gdbs_ref[...]
        acc_scr[0] = 0.0
        acc_scr[1] = 0.0

    t = jnp.concatenate([z1_ref[...], z2_ref[...], z3_ref[...]], axis=1)
    hl = t
    for j in range(ldW_ref.shape[0]):
        hl = jnp.maximum(
            jnp.dot(hl, ldW_ref[j], preferred_element_type=jnp.float32)
            + ldb_ref[j], 0.0)
    l_enc = hl + jnp.dot(
        t, ldWs_ref[...], preferred_element_type=jnp.float32
    ) + ldbs_ref[...]
    res = lax.dot_general(l_enc, genc_scr[...], (((1,), (1,)), ((), ())),
                          preferred_element_type=jnp.float32)
    posm = (batchf_ref[...] == lax.broadcasted_iota(jnp.int32, (nb, g), 1)
            .astype(jnp.float32))
    sp = jnp.maximum(-res, 0.0) + jnp.log(1.0 + jnp.exp(-jnp.abs(res)))
    pos_t = jnp.where(posm, _LOG2 - sp, 0.0)
    neg_t = jnp.where(posm, 0.0, sp + res - _LOG2)
    acc_scr[0] += jnp.sum(pos_t)
    acc_scr[1] += jnp.sum(neg_t)

    @pl.when(i == nblk - 1)
    def _():
        n_total = nb * nblk
        out_ref[0] = acc_scr[1] / (n_total * (g - 1)) - acc_scr[0] / n_total


def _heads_loss(y, gd_W, gd_b, gd_Ws, gd_bs, ld_W, ld_b, ld_Ws, ld_bs,
                z1, z2, z3, batchf):
    n, d = z1.shape
    g, emb = y.shape
    nblk = 10
    nb = n // nblk
    full = lambda shape: pl.BlockSpec(shape, lambda i: (0,) * len(shape))
    out = pl.pallas_call(
        functools.partial(_head_body, nblk),
        grid=(nblk,),
        in_specs=[
            full((g, emb)),
            full(gd_W.shape), full((gd_W.shape[0], 1, emb)), full(gd_Ws.shape),
            full((1, emb)),
            full(ld_W.shape), full((ld_W.shape[0], 1, emb)), full(ld_Ws.shape),
            full((1, emb)),
            pl.BlockSpec((nb, d), lambda i: (i, 0)),
            pl.BlockSpec((nb, d), lambda i: (i, 0)),
            pl.BlockSpec((nb, d), lambda i: (i, 0)),
            pl.BlockSpec((nb, 1), lambda i: (i, 0)),
        ],
        out_specs=pl.BlockSpec(memory_space=pltpu.SMEM),
        out_shape=jax.ShapeDtypeStruct((1,), jnp.float32),
        scratch_shapes=[
            pltpu.VMEM((g, emb), jnp.float32),
            pltpu.SMEM((2,), jnp.float32),
        ],
    )(y, gd_W, gd_b.reshape(-1, 1, emb), gd_Ws, gd_bs.reshape(1, emb),
      ld_W, ld_b.reshape(-1, 1, emb), ld_Ws, ld_bs.reshape(1, emb),
      z1, z2, z3, batchf)
    return out[0]


def kernel(x, edge_index, batch, num_graphs, conv_W1, conv_b1, conv_W2,
           conv_b2, bn_gamma, bn_beta, ld_W, ld_b, ld_Ws, ld_bs, gd_W, gd_b,
           gd_Ws, gd_bs):
    n, d = x.shape
    g = 128
    npad = _EPAD - _E
    pad_i = jnp.arange(npad, dtype=jnp.int32)
    src_p = jnp.concatenate(
        [edge_index[0], pad_i % _N]).reshape(-1, _CH)
    dst_p = jnp.concatenate(
        [edge_index[1], _N + pad_i % _NDUMP]).reshape(-1, _CH)
    batchf = batch.astype(jnp.float32).reshape(n, 1)

    zeros = jnp.zeros((n, d), jnp.float32)
    h = x
    zs = []
    pools = []
    for i in range(conv_W1.shape[0]):
        agg = _edge_agg(h, src_p, dst_p, zeros)
        h, pool = _dense_layer(h, agg, conv_W1[i], conv_b1[i], conv_W2[i],
                               conv_b2[i], bn_gamma[i], bn_beta[i], batchf, g)
        zs.append(h)
        pools.append(pool)
    y = jnp.concatenate(pools, axis=1)
    return _heads_loss(y, gd_W, gd_b, gd_Ws, gd_bs, ld_W, ld_b, ld_Ws, ld_bs,
                       zs[0], zs[1], zs[2], batchf)


# speedup vs baseline: 1.2494x; 1.0177x over previous
import functools

import jax
import jax.numpy as jnp
from jax import lax
from jax.experimental import pallas as pl
from jax.experimental.pallas import tpu as pltpu
from jax.experimental.pallas import tpu_sc as plsc

_LOG2 = 0.6931471805599453

_N = 10000
_D = 128
_E = 320000
_NC = 2
_NS = 16
_NW = _NC * _NS
_CH = 64
_CPT = 160
_CPP = 16
_RING = 4
_EPW = _CH * _CPT
_EPAD = _NW * _EPW
_NDUMP = 768
_NACC = _N + _NDUMP
_RPT = 624
_RTAIL = _N - _RPT * _NS


def _edge_agg_body(h_hbm, src_hbm, dst_hbm, zeros_hbm, out_hbm,
                   src_a, dst_a, src_b, dst_b, *bufs):
    rows = bufs[:_RING]
    acc_sh = bufs[_RING]
    isem = bufs[_RING + 1]
    gsem = bufs[_RING + 2:2 * _RING + 2]
    ssem = bufs[2 * _RING + 2:]
    idx = ((src_a, dst_a), (src_b, dst_b))
    cid = lax.axis_index("c")
    sid = lax.axis_index("s")
    wid = cid * _NS + sid

    def wait_gather(sv, c, j):
        pltpu.make_async_copy(h_hbm.at[sv.at[c]], rows[j], gsem[j]).wait()

    def scatter(dv, c, j):
        pltpu.async_copy(rows[j], acc_sh.at[dv.at[c]], ssem[j], add=True)

    def wait_scatter(dv, c, j):
        pltpu.make_async_copy(rows[j], acc_sh.at[dv.at[c]], ssem[j]).wait()

    def gather(sv, c, j):
        pltpu.async_copy(h_hbm.at[sv.at[c]], rows[j], gsem[j])

    def load_idx(p, sync):
        sv, dv = idx[p % 2]
        cbase = wid * _CPT + p * _CPP
        if sync:
            pltpu.sync_copy(src_hbm.at[pl.ds(cbase, _CPP), :], sv)
            pltpu.sync_copy(dst_hbm.at[pl.ds(cbase, _CPP), :], dv)
        else:
            pltpu.async_copy(src_hbm.at[pl.ds(cbase, _CPP), :], sv, isem)
            pltpu.async_copy(dst_hbm.at[pl.ds(cbase, _CPP), :], dv, isem)

    def wait_idx(p):
        sv, dv = idx[p % 2]
        cbase = wid * _CPT + p * _CPP
        pltpu.make_async_copy(src_hbm.at[pl.ds(cbase, _CPP), :], sv,
                              isem).wait()
        pltpu.make_async_copy(dst_hbm.at[pl.ds(cbase, _CPP), :], dv,
                              isem).wait()

    nphase = _CPT // _CPP
    ngrp = _CPP // _RING
    load_idx(0, sync=True)
    for j in range(_RING):
        gather(idx[0][0], j, j)
    rz = sid * _RPT
    pltpu.sync_copy(zeros_hbm.at[pl.ds(rz, _RPT), :],
                    acc_sh.at[pl.ds(rz, _RPT), :])

    @pl.when(sid == 0)
    def _():
        pltpu.sync_copy(zeros_hbm.at[pl.ds(_RPT * _NS, _RTAIL), :],
                        acc_sh.at[pl.ds(_RPT * _NS, _RTAIL), :])

    plsc.subcore_barrier()
    for p in range(nphase):
        sv, dv = idx[p % 2]
        if p + 1 < nphase:
            load_idx(p + 1, sync=False)

        def body(k, carry):
            c0 = _RING * k
            for j in range(_RING):
                wait_gather(sv, c0 + j, j)
                scatter(dv, c0 + j, j)
            for j in range(_RING):
                wait_scatter(dv, c0 + j, j)
                gather(sv, c0 + _RING + j, j)
            return carry

        lax.fori_loop(0, ngrp - 1, body, 0)
        cl = _CPP - _RING
        for j in range(_RING):
            wait_gather(sv, cl + j, j)
            scatter(dv, cl + j, j)
        if p + 1 < nphase:
            wait_idx(p + 1)
            nsv = idx[(p + 1) % 2][0]
            for j in range(_RING):
                wait_scatter(dv, cl + j, j)
                gather(nsv, j, j)
        else:
            for j in range(_RING):
                wait_scatter(dv, cl + j, j)

    plsc.subcore_barrier()
    pltpu.sync_copy(acc_sh.at[pl.ds(rz, _RPT), :],
                    out_hbm.at[cid, pl.ds(rz, _RPT), :])

    @pl.when(sid == 0)
    def _():
        pltpu.sync_copy(acc_sh.at[pl.ds(_RPT * _NS, _RTAIL), :],
                        out_hbm.at[cid, pl.ds(_RPT * _NS, _RTAIL), :])


def _edge_agg(h, src_p, dst_p, zeros):
    mesh = plsc.VectorSubcoreMesh(core_axis_name="c", subcore_axis_name="s")
    f = pl.kernel(
        _edge_agg_body, mesh=mesh,
        out_type=jax.ShapeDtypeStruct((_NC, _N, _D), jnp.float32),
        scratch_types=(
            [pltpu.VMEM((_CPP, _CH), jnp.int32)] * 4
            + [pltpu.VMEM((_CH, _D), jnp.float32)] * _RING
            + [pltpu.VMEM_SHARED((_NACC, _D), jnp.float32)]
            + [pltpu.SemaphoreType.DMA] * (1 + 2 * _RING)
        ),
    )
    return f(h, src_p, dst_p, zeros)


def _layer_body(h_ref, agg_ref, w1_ref, b1_ref, w2_ref, b2_ref, gam_ref,
                bet_ref, batchf_ref, z_ref, pool_ref):
    n, d = h_ref.shape
    g = pool_ref.shape[0]
    z0 = h_ref[...] + agg_ref[0] + agg_ref[1]
    a = jnp.maximum(
        jnp.dot(z0, w1_ref[...], preferred_element_type=jnp.float32)
        + b1_ref[...], 0.0)
    u = jnp.maximum(
        jnp.dot(a, w2_ref[...], preferred_element_type=jnp.float32)
        + b2_ref[...], 0.0)
    mu = jnp.mean(u, axis=0, keepdims=True)
    var = jnp.mean((u - mu) ** 2, axis=0, keepdims=True)
    zn = (u - mu) * (gam_ref[...] * lax.rsqrt(var + 1e-5)) + bet_ref[...]
    z_ref[...] = zn
    onehot = (batchf_ref[...] == lax.broadcasted_iota(jnp.int32, (n, g), 1)
              .astype(jnp.float32)).astype(jnp.float32)
    pool_ref[...] = lax.dot_general(
        onehot, zn, (((0,), (0,)), ((), ())),
        preferred_element_type=jnp.float32)


def _dense_layer(h, agg, w1, b1, w2, b2, gam, bet, batchf, g):
    n, d = h.shape
    return pl.pallas_call(
        _layer_body,
        out_shape=(jax.ShapeDtypeStruct((n, d), jnp.float32),
                   jax.ShapeDtypeStruct((g, d), jnp.float32)),
    )(h, agg, w1, b1.reshape(1, d), w2, b2.reshape(1, d),
      gam.reshape(1, d), bet.reshape(1, d), batchf)


def _head_body(nblk, y_ref, gdW_ref, gdb_ref, gdWs_ref, gdbs_ref, ldW_ref,
               ldb_ref, ldWs_ref, ldbs_ref, z1_ref, z2_ref, z3_ref,
               batchf_ref, out_ref, genc_scr, acc_scr):
    i = pl.program_id(0)
    nb = z1_ref.shape[0]
    g, emb = genc_scr.shape

    @pl.when(i == 0)
    def _():
        y = y_ref[...]
        hg = y
        for j in range(gdW_ref.shape[0]):
            hg = jnp.maximum(
                jnp.dot(hg, gdW_ref[j], preferred_element_type=jnp.float32)
                + gdb_ref[j], 0.0)
        genc_scr[...] = hg + jnp.dot(
            y, gdWs_ref[...], preferred_element_type=jnp.float32
        ) + gdbs_ref[...]
        acc_scr[0] = 0.0
        acc_scr[1] = 0.0

    t = jnp.concatenate([z1_ref[...], z2_ref[...], z3_ref[...]], axis=1)
    hl = t
    for j in range(ldW_ref.shape[0]):
        hl = jnp.maximum(
            jnp.dot(hl, ldW_ref[j], preferred_element_type=jnp.float32)
            + ldb_ref[j], 0.0)
    l_enc = hl + jnp.dot(
        t, ldWs_ref[...], preferred_element_type=jnp.float32
    ) + ldbs_ref[...]
    res = lax.dot_general(l_enc, genc_scr[...], (((1,), (1,)), ((), ())),
                          preferred_element_type=jnp.float32)
    posm = (batchf_ref[...] == lax.broadcasted_iota(jnp.int32, (nb, g), 1)
            .astype(jnp.float32))
    sp = jnp.maximum(-res, 0.0) + jnp.log(1.0 + jnp.exp(-jnp.abs(res)))
    pos_t = jnp.where(posm, _LOG2 - sp, 0.0)
    neg_t = jnp.where(posm, 0.0, sp + res - _LOG2)
    acc_scr[0] += jnp.sum(pos_t)
    acc_scr[1] += jnp.sum(neg_t)

    @pl.when(i == nblk - 1)
    def _():
        n_total = nb * nblk
        out_ref[0] = acc_scr[1] / (n_total * (g - 1)) - acc_scr[0] / n_total


def _heads_loss(y, gd_W, gd_b, gd_Ws, gd_bs, ld_W, ld_b, ld_Ws, ld_bs,
                z1, z2, z3, batchf):
    n, d = z1.shape
    g, emb = y.shape
    nblk = 10
    nb = n // nblk
    full = lambda shape: pl.BlockSpec(shape, lambda i: (0,) * len(shape))
    out = pl.pallas_call(
        functools.partial(_head_body, nblk),
        grid=(nblk,),
        in_specs=[
            full((g, emb)),
            full(gd_W.shape), full((gd_W.shape[0], 1, emb)), full(gd_Ws.shape),
            full((1, emb)),
            full(ld_W.shape), full((ld_W.shape[0], 1, emb)), full(ld_Ws.shape),
            full((1, emb)),
            pl.BlockSpec((nb, d), lambda i: (i, 0)),
            pl.BlockSpec((nb, d), lambda i: (i, 0)),
            pl.BlockSpec((nb, d), lambda i: (i, 0)),
            pl.BlockSpec((nb, 1), lambda i: (i, 0)),
        ],
        out_specs=pl.BlockSpec(memory_space=pltpu.SMEM),
        out_shape=jax.ShapeDtypeStruct((1,), jnp.float32),
        scratch_shapes=[
            pltpu.VMEM((g, emb), jnp.float32),
            pltpu.SMEM((2,), jnp.float32),
        ],
    )(y, gd_W, gd_b.reshape(-1, 1, emb), gd_Ws, gd_bs.reshape(1, emb),
      ld_W, ld_b.reshape(-1, 1, emb), ld_Ws, ld_bs.reshape(1, emb),
      z1, z2, z3, batchf)
    return out[0]


def kernel(x, edge_index, batch, num_graphs, conv_W1, conv_b1, conv_W2,
           conv_b2, bn_gamma, bn_beta, ld_W, ld_b, ld_Ws, ld_bs, gd_W, gd_b,
           gd_Ws, gd_bs):
    n, d = x.shape
    g = 128
    npad = _EPAD - _E
    pad_i = jnp.arange(npad, dtype=jnp.int32)
    src_p = jnp.concatenate(
        [edge_index[0], pad_i % _N]).reshape(-1, _CH)
    dst_p = jnp.concatenate(
        [edge_index[1], _N + pad_i % _NDUMP]).reshape(-1, _CH)
    batchf = batch.astype(jnp.float32).reshape(n, 1)

    zeros = jnp.zeros((n, d), jnp.float32)
    h = x
    zs = []
    pools = []
    for i in range(conv_W1.shape[0]):
        agg = _edge_agg(h, src_p, dst_p, zeros)
        h, pool = _dense_layer(h, agg, conv_W1[i], conv_b1[i], conv_W2[i],
                               conv_b2[i], bn_gamma[i], bn_beta[i], batchf, g)
        zs.append(h)
        pools.append(pool)
    y = jnp.concatenate(pools, axis=1)
    return _heads_loss(y, gd_W, gd_b, gd_Ws, gd_bs, ld_W, ld_b, ld_Ws, ld_bs,
                       zs[0], zs[1], zs[2], batchf)
